# baseline (speedup 1.0000x reference)
import functools

import jax
import jax.numpy as jnp
from jax import lax
from jax.experimental import pallas as pl
from jax.experimental.pallas import tpu as pltpu
from jax.experimental.pallas import tpu_sc as plsc

B = 4096
V = 1000000
D = 16
L = 200

NC = 2
NS = 16
NW = NC * NS
BPW = B // NW
NBUF = 4


def _hist_body(hidx_hbm, wh_hbm, out_hbm, hidx_v, b0, b1, b2, b3, out_v,
               s0, s1, s2, s3):
    wid = lax.axis_index("s") * NC + lax.axis_index("c")
    base = wid * BPW
    bufs = (b0, b1, b2, b3)
    sems = (s0, s1, s2, s3)

    pltpu.sync_copy(hidx_hbm.at[pl.ds(2 * base, 2 * BPW)], hidx_v)

    def fire(e, b):
        pltpu.async_copy(wh_hbm.at[hidx_v.at[2 * e]],
                         bufs[b].at[pl.ds(0, L // 2)], sems[b])
        pltpu.async_copy(wh_hbm.at[hidx_v.at[2 * e + 1]],
                         bufs[b].at[pl.ds(L // 2, L // 2)], sems[b])

    for b in range(NBUF):
        fire(b, b)

    inv_l = jnp.float32(1.0 / L)

    def body(k, _):
        for b in range(NBUF):
            e = NBUF * k + b
            pltpu.make_async_copy(wh_hbm.at[hidx_v.at[2 * e]],
                                  bufs[b].at[pl.ds(0, L // 2)],
                                  sems[b]).wait()
            pltpu.make_async_copy(wh_hbm.at[hidx_v.at[2 * e + 1]],
                                  bufs[b].at[pl.ds(L // 2, L // 2)],
                                  sems[b]).wait()
            buf = bufs[b]
            acc0 = buf[0, :]
            acc1 = buf[1, :]
            acc2 = buf[2, :]
            acc3 = buf[3, :]
            for j in range(4, L, 4):
                acc0 = acc0 + buf[j, :]
                acc1 = acc1 + buf[j + 1, :]
                acc2 = acc2 + buf[j + 2, :]
                acc3 = acc3 + buf[j + 3, :]
            total = (acc0 + acc1) + (acc2 + acc3)
            out_v[e, :] = total * inv_l

            @pl.when(k < BPW // NBUF - 1)
            def _():
                fire(e + NBUF, b)
        return None

    lax.fori_loop(0, BPW // NBUF, body, None)

    pltpu.sync_copy(out_v, out_hbm.at[pl.ds(base, BPW)])


def _ui_body(uid_hbm, iid_hbm, wut_hbm, wit_hbm, out_hbm,
             uidx_v, iidx_v, slab_u, slab_i, out_v, su, si):
    wid = lax.axis_index("s") * NC + lax.axis_index("c")
    base = wid * BPW

    pltpu.sync_copy(uid_hbm.at[pl.ds(base, BPW)], uidx_v)
    pltpu.sync_copy(iid_hbm.at[pl.ds(base, BPW)], iidx_v)

    uvecs = [uidx_v[pl.ds(g * 16, 16)] for g in range(BPW // 16)]
    ivecs = [iidx_v[pl.ds(g * 16, 16)] for g in range(BPW // 16)]
    NSL = 4

    SW = 128

    def col_base(v):
        return pl.multiple_of((v // SW) * SW, SW)

    def refs(i, b):
        cu = col_base(uvecs[i // 16][i % 16])
        ci = col_base(ivecs[i // 16][i % 16])
        return (pltpu.make_async_copy(wut_hbm.at[:, pl.ds(cu, SW)],
                                      slab_u.at[b], su.at[b]),
                pltpu.make_async_copy(wit_hbm.at[:, pl.ds(ci, SW)],
                                      slab_i.at[b], si.at[b]))

    for b in range(NSL):
        for c in refs(b, b):
            c.start()

    lanes = lax.iota(jnp.int32, D)

    for i in range(BPW):
        b = i % NSL
        for c in refs(i, b):
            c.wait()
        u = uvecs[i // 16][i % 16]
        v = ivecs[i // 16][i % 16]
        colu = u - col_base(u)
        coli = v - col_base(v)
        vu = plsc.load_gather(slab_u.at[b],
                              [lanes, jnp.full((D,), colu, jnp.int32)])
        vi = plsc.load_gather(slab_i.at[b],
                              [lanes, jnp.full((D,), coli, jnp.int32)])
        out_v[i, pl.ds(0, D)] = vu
        out_v[i, pl.ds(D, D)] = vi
        if i + NSL < BPW:
            for c in refs(i + NSL, b):
                c.start()

    pltpu.sync_copy(out_v, out_hbm.at[pl.ds(base, BPW)])


_TRBW = 1024


def _tr_body(in_ref, out_ref):
    x = in_ref[...]
    out_ref[...] = x.reshape(D, _TRBW // 8, 8).transpose(1, 2, 0).reshape(
        _TRBW // 8, 8 * D)


@jax.jit
def _run(uid, iid, hidx, wu, wi, wh):
    wh_lin = pl.pallas_call(
        _tr_body,
        grid=(pl.cdiv(V, _TRBW),),
        in_specs=[pl.BlockSpec((D, _TRBW), lambda q: (0, q))],
        out_specs=pl.BlockSpec((_TRBW // 8, 8 * D), lambda q: (q, 0)),
        out_shape=jax.ShapeDtypeStruct((V // 8, 8 * D), jnp.float32),
    )(wh.T)
    wh_lin = wh_lin.reshape(V, D)
    mesh = plsc.VectorSubcoreMesh(
        core_axis_name="c", subcore_axis_name="s",
        num_cores=NC, num_subcores=NS)

    hist_out = pl.kernel(
        _hist_body,
        out_type=jax.ShapeDtypeStruct((B, D), jnp.float32),
        mesh=mesh,
        compiler_params=pltpu.CompilerParams(use_tc_tiling_on_sc=False),
        scratch_types=[
            pltpu.VMEM((2 * BPW, L // 2), jnp.int32),
            pltpu.VMEM((L, D), jnp.float32),
            pltpu.VMEM((L, D), jnp.float32),
            pltpu.VMEM((L, D), jnp.float32),
            pltpu.VMEM((L, D), jnp.float32),
            pltpu.VMEM((BPW, D), jnp.float32),
            pltpu.SemaphoreType.DMA,
            pltpu.SemaphoreType.DMA,
            pltpu.SemaphoreType.DMA,
            pltpu.SemaphoreType.DMA,
        ],
    )(hidx.reshape(2 * B, L // 2), wh_lin)

    ui_out = pl.kernel(
        _ui_body,
        out_type=jax.ShapeDtypeStruct((B, 2 * D), jnp.float32),
        mesh=mesh,
        compiler_params=pltpu.CompilerParams(needs_layout_passes=False,
                                             disable_bounds_checks=True),
        scratch_types=[
            pltpu.VMEM((BPW,), jnp.int32),
            pltpu.VMEM((BPW,), jnp.int32),
            pltpu.VMEM((4, D, 128), jnp.float32),
            pltpu.VMEM((4, D, 128), jnp.float32),
            pltpu.VMEM((BPW, 2 * D), jnp.float32),
            pltpu.SemaphoreType.DMA((4,)),
            pltpu.SemaphoreType.DMA((4,)),
        ],
    )(uid, iid, wu.T, wi.T)

    return jnp.concatenate([hist_out, ui_out], axis=1)


def kernel(user_id, item_id, hist_item, W_user, W_item, W_hist):
    uid = user_id.reshape(B)
    iid = item_id.reshape(B)
    return _run(uid, iid, hist_item, W_user, W_item, W_hist)


# speedup vs baseline: 1.2444x; 1.2444x over previous
import functools

import jax
import jax.numpy as jnp
from jax import lax
from jax.experimental import pallas as pl
from jax.experimental.pallas import tpu as pltpu
from jax.experimental.pallas import tpu_sc as plsc

B = 4096
V = 1000000
D = 16
L = 200

NC = 2
NS = 16
NW = NC * NS
BPW = B // NW
NBUF = 4


def _hist_body(hidx_hbm, wh_hbm, out_hbm, hidx_v, b0, b1, b2, b3, out_v,
               s0, s1, s2, s3):
    wid = lax.axis_index("s") * NC + lax.axis_index("c")
    base = wid * BPW
    bufs = (b0, b1, b2, b3)
    sems = (s0, s1, s2, s3)

    pltpu.sync_copy(hidx_hbm.at[pl.ds(2 * base, 2 * BPW)], hidx_v)

    def fire(e, b):
        pltpu.async_copy(wh_hbm.at[hidx_v.at[2 * e]],
                         bufs[b].at[pl.ds(0, L // 2)], sems[b])
        pltpu.async_copy(wh_hbm.at[hidx_v.at[2 * e + 1]],
                         bufs[b].at[pl.ds(L // 2, L // 2)], sems[b])

    for b in range(NBUF):
        fire(b, b)

    inv_l = jnp.float32(1.0 / L)

    def body(k, _):
        for b in range(NBUF):
            e = NBUF * k + b
            pltpu.make_async_copy(wh_hbm.at[hidx_v.at[2 * e]],
                                  bufs[b].at[pl.ds(0, L // 2)],
                                  sems[b]).wait()
            pltpu.make_async_copy(wh_hbm.at[hidx_v.at[2 * e + 1]],
                                  bufs[b].at[pl.ds(L // 2, L // 2)],
                                  sems[b]).wait()
            buf = bufs[b]
            acc0 = buf[0, :]
            acc1 = buf[1, :]
            acc2 = buf[2, :]
            acc3 = buf[3, :]
            for j in range(4, L, 4):
                acc0 = acc0 + buf[j, :]
                acc1 = acc1 + buf[j + 1, :]
                acc2 = acc2 + buf[j + 2, :]
                acc3 = acc3 + buf[j + 3, :]
            total = (acc0 + acc1) + (acc2 + acc3)
            out_v[e, :] = total * inv_l

            @pl.when(k < BPW // NBUF - 1)
            def _():
                fire(e + NBUF, b)
        return None

    lax.fori_loop(0, BPW // NBUF, body, None)

    pltpu.sync_copy(out_v, out_hbm.at[pl.ds(base, BPW)])


def _ui_body(uid_hbm, iid_hbm, wut_hbm, wit_hbm, out_hbm,
             uidx_v, iidx_v, slab_u, slab_i, out_v, su, si):
    wid = lax.axis_index("s") * NC + lax.axis_index("c")
    base = wid * BPW

    pltpu.sync_copy(uid_hbm.at[pl.ds(base, BPW)], uidx_v)
    pltpu.sync_copy(iid_hbm.at[pl.ds(base, BPW)], iidx_v)

    uvecs = [uidx_v[pl.ds(g * 16, 16)] for g in range(BPW // 16)]
    ivecs = [iidx_v[pl.ds(g * 16, 16)] for g in range(BPW // 16)]
    NSL = 4

    SW = 128

    def col_base(v):
        return pl.multiple_of((v // SW) * SW, SW)

    def refs(i, b):
        cu = col_base(uvecs[i // 16][i % 16])
        ci = col_base(ivecs[i // 16][i % 16])
        return (pltpu.make_async_copy(wut_hbm.at[:, pl.ds(cu, SW)],
                                      slab_u.at[b], su.at[b]),
                pltpu.make_async_copy(wit_hbm.at[:, pl.ds(ci, SW)],
                                      slab_i.at[b], si.at[b]))

    for b in range(NSL):
        for c in refs(b, b):
            c.start()

    lanes = lax.iota(jnp.int32, D)

    for i in range(BPW):
        b = i % NSL
        for c in refs(i, b):
            c.wait()
        u = uvecs[i // 16][i % 16]
        v = ivecs[i // 16][i % 16]
        colu = u - col_base(u)
        coli = v - col_base(v)
        vu = plsc.load_gather(slab_u.at[b],
                              [lanes, jnp.full((D,), colu, jnp.int32)])
        vi = plsc.load_gather(slab_i.at[b],
                              [lanes, jnp.full((D,), coli, jnp.int32)])
        out_v[i, pl.ds(0, D)] = vu
        out_v[i, pl.ds(D, D)] = vi
        if i + NSL < BPW:
            for c in refs(i + NSL, b):
                c.start()

    pltpu.sync_copy(out_v, out_hbm.at[pl.ds(base, BPW)])


@jax.jit
def _run(uid, iid, hidx, wu, wi, wh):
    wh_lin = -lax.optimization_barrier(-wh)
    mesh = plsc.VectorSubcoreMesh(
        core_axis_name="c", subcore_axis_name="s",
        num_cores=NC, num_subcores=NS)

    hist_out = pl.kernel(
        _hist_body,
        out_type=jax.ShapeDtypeStruct((B, D), jnp.float32),
        mesh=mesh,
        compiler_params=pltpu.CompilerParams(use_tc_tiling_on_sc=False),
        scratch_types=[
            pltpu.VMEM((2 * BPW, L // 2), jnp.int32),
            pltpu.VMEM((L, D), jnp.float32),
            pltpu.VMEM((L, D), jnp.float32),
            pltpu.VMEM((L, D), jnp.float32),
            pltpu.VMEM((L, D), jnp.float32),
            pltpu.VMEM((BPW, D), jnp.float32),
            pltpu.SemaphoreType.DMA,
            pltpu.SemaphoreType.DMA,
            pltpu.SemaphoreType.DMA,
            pltpu.SemaphoreType.DMA,
        ],
    )(hidx.reshape(2 * B, L // 2), wh_lin)

    ui_out = pl.kernel(
        _ui_body,
        out_type=jax.ShapeDtypeStruct((B, 2 * D), jnp.float32),
        mesh=mesh,
        compiler_params=pltpu.CompilerParams(needs_layout_passes=False,
                                             disable_bounds_checks=True),
        scratch_types=[
            pltpu.VMEM((BPW,), jnp.int32),
            pltpu.VMEM((BPW,), jnp.int32),
            pltpu.VMEM((4, D, 128), jnp.float32),
            pltpu.VMEM((4, D, 128), jnp.float32),
            pltpu.VMEM((BPW, 2 * D), jnp.float32),
            pltpu.SemaphoreType.DMA((4,)),
            pltpu.SemaphoreType.DMA((4,)),
        ],
    )(uid, iid, wu.T, wi.T)

    return jnp.concatenate([hist_out, ui_out], axis=1)


def kernel(user_id, item_id, hist_item, W_user, W_item, W_hist):
    uid = user_id.reshape(B)
    iid = item_id.reshape(B)
    return _run(uid, iid, hist_item, W_user, W_item, W_hist)


# speedup vs baseline: 2.1670x; 1.7414x over previous
import functools

import jax
import jax.numpy as jnp
from jax import lax
from jax.experimental import pallas as pl
from jax.experimental.pallas import tpu as pltpu
from jax.experimental.pallas import tpu_sc as plsc

B = 4096
V = 1000000
D = 16
L = 200

NC = 2
NS = 16
NW = NC * NS
BPW = B // NW
NBUF = 4


def _hist_body(hidx_hbm, wh_hbm, out_hbm, hidx_v, b0, b1, b2, b3, out_v,
               s0, s1, s2, s3):
    wid = lax.axis_index("s") * NC + lax.axis_index("c")
    base = wid * BPW
    bufs = (b0, b1, b2, b3)
    sems = (s0, s1, s2, s3)

    pltpu.sync_copy(hidx_hbm.at[pl.ds(2 * base, 2 * BPW)], hidx_v)

    def fire(e, b):
        pltpu.async_copy(wh_hbm.at[hidx_v.at[2 * e]],
                         bufs[b].at[pl.ds(0, L // 2)], sems[b])
        pltpu.async_copy(wh_hbm.at[hidx_v.at[2 * e + 1]],
                         bufs[b].at[pl.ds(L // 2, L // 2)], sems[b])

    for b in range(NBUF):
        fire(b, b)

    inv_l = jnp.float32(1.0 / L)

    def body(k, _):
        for b in range(NBUF):
            e = NBUF * k + b
            pltpu.make_async_copy(wh_hbm.at[hidx_v.at[2 * e]],
                                  bufs[b].at[pl.ds(0, L // 2)],
                                  sems[b]).wait()
            pltpu.make_async_copy(wh_hbm.at[hidx_v.at[2 * e + 1]],
                                  bufs[b].at[pl.ds(L // 2, L // 2)],
                                  sems[b]).wait()
            buf = bufs[b]

            def red(j, accs):
                r = j * 8
                return tuple(accs[t] + buf[r + 2 * t, :] +
                             buf[r + 2 * t + 1, :] for t in range(4))

            z = jnp.zeros((D,), jnp.float32)
            accs = lax.fori_loop(0, L // 8, red, (z, z, z, z))
            total = (accs[0] + accs[1]) + (accs[2] + accs[3])
            out_v[e, :] = total * inv_l

            @pl.when(k < BPW // NBUF - 1)
            def _():
                fire(e + NBUF, b)
        return None

    lax.fori_loop(0, BPW // NBUF, body, None)

    pltpu.sync_copy(out_v, out_hbm.at[pl.ds(base, BPW)])


def _ui_body(uid_hbm, iid_hbm, wut_hbm, wit_hbm, out_hbm,
             uidx_v, iidx_v, slab_u, slab_i, out_v, su, si):
    wid = lax.axis_index("s") * NC + lax.axis_index("c")
    base = wid * BPW

    pltpu.sync_copy(uid_hbm.at[pl.ds(base, BPW)], uidx_v)
    pltpu.sync_copy(iid_hbm.at[pl.ds(base, BPW)], iidx_v)

    uvecs = [uidx_v[pl.ds(g * 16, 16)] for g in range(BPW // 16)]
    ivecs = [iidx_v[pl.ds(g * 16, 16)] for g in range(BPW // 16)]
    NSL = 4

    SW = 128

    def col_base(v):
        return pl.multiple_of((v // SW) * SW, SW)

    def refs(i, b):
        cu = col_base(uvecs[i // 16][i % 16])
        ci = col_base(ivecs[i // 16][i % 16])
        return (pltpu.make_async_copy(wut_hbm.at[:, pl.ds(cu, SW)],
                                      slab_u.at[b], su.at[b]),
                pltpu.make_async_copy(wit_hbm.at[:, pl.ds(ci, SW)],
                                      slab_i.at[b], si.at[b]))

    for b in range(NSL):
        for c in refs(b, b):
            c.start()

    lanes = lax.iota(jnp.int32, D)

    for i in range(BPW):
        b = i % NSL
        for c in refs(i, b):
            c.wait()
        u = uvecs[i // 16][i % 16]
        v = ivecs[i // 16][i % 16]
        colu = u - col_base(u)
        coli = v - col_base(v)
        vu = plsc.load_gather(slab_u.at[b],
                              [lanes, jnp.full((D,), colu, jnp.int32)])
        vi = plsc.load_gather(slab_i.at[b],
                              [lanes, jnp.full((D,), coli, jnp.int32)])
        out_v[i, pl.ds(0, D)] = vu
        out_v[i, pl.ds(D, D)] = vi
        if i + NSL < BPW:
            for c in refs(i + NSL, b):
                c.start()

    pltpu.sync_copy(out_v, out_hbm.at[pl.ds(base, BPW)])


@jax.jit
def _run(uid, iid, hidx, wu, wi, wh):
    mesh = plsc.VectorSubcoreMesh(
        core_axis_name="c", subcore_axis_name="s",
        num_cores=NC, num_subcores=NS)

    hist_out = pl.kernel(
        _hist_body,
        out_type=jax.ShapeDtypeStruct((B, D), jnp.float32),
        mesh=mesh,
        compiler_params=pltpu.CompilerParams(use_tc_tiling_on_sc=False),
        scratch_types=[
            pltpu.VMEM((2 * BPW, L // 2), jnp.int32),
            pltpu.VMEM((L, D), jnp.float32),
            pltpu.VMEM((L, D), jnp.float32),
            pltpu.VMEM((L, D), jnp.float32),
            pltpu.VMEM((L, D), jnp.float32),
            pltpu.VMEM((BPW, D), jnp.float32),
            pltpu.SemaphoreType.DMA,
            pltpu.SemaphoreType.DMA,
            pltpu.SemaphoreType.DMA,
            pltpu.SemaphoreType.DMA,
        ],
    )(hidx.reshape(2 * B, L // 2), wh)

    ui_out = pl.kernel(
        _ui_body,
        out_type=jax.ShapeDtypeStruct((B, 2 * D), jnp.float32),
        mesh=mesh,
        compiler_params=pltpu.CompilerParams(needs_layout_passes=False,
                                             disable_bounds_checks=True),
        scratch_types=[
            pltpu.VMEM((BPW,), jnp.int32),
            pltpu.VMEM((BPW,), jnp.int32),
            pltpu.VMEM((4, D, 128), jnp.float32),
            pltpu.VMEM((4, D, 128), jnp.float32),
            pltpu.VMEM((BPW, 2 * D), jnp.float32),
            pltpu.SemaphoreType.DMA((4,)),
            pltpu.SemaphoreType.DMA((4,)),
        ],
    )(uid, iid, wu.T, wi.T)

    return jnp.concatenate([hist_out, ui_out], axis=1)


def kernel(user_id, item_id, hist_item, W_user, W_item, W_hist):
    uid = user_id.reshape(B)
    iid = item_id.reshape(B)
    return _run(uid, iid, hist_item, W_user, W_item, W_hist)
